# baseline (speedup 1.0000x reference)
import functools

import jax
import jax.numpy as jnp
from jax import lax
from jax.experimental import pallas as pl
from jax.experimental.pallas import tpu as pltpu
from jax.experimental.pallas import tpu_sc as plsc

NC = 2
NS = 16
NW = NC * NS
L = 16

NG = 4
NO = 2
D = 32


@functools.lru_cache(maxsize=None)
def _build_conv_in(vocab, d):
    assert d == D
    grp = 128 // d
    VB = 512
    n_steps = -(-(vocab // grp) // VB)

    def body(t_ref, o_ref):
        a = t_ref[...]
        a3 = a.reshape(d, VB, grp)
        o_ref[...] = a3.transpose(1, 2, 0).reshape(VB, 128)

    return pl.pallas_call(
        body,
        grid=(n_steps,),
        in_specs=[pl.BlockSpec((d, grp * VB), lambda g: (0, g))],
        out_specs=pl.BlockSpec((VB, 128), lambda g: (g, 0)),
        out_shape=jax.ShapeDtypeStruct((vocab // grp, 128), jnp.float32),
    )


@functools.lru_cache(maxsize=None)
def _build_emb(batch, seq, vocab, d):
    assert d == D
    rows_per_w = batch // NW
    n_groups = rows_per_w // NG
    assert rows_per_w % NG == 0
    row128 = seq * d // 128
    mesh = plsc.VectorSubcoreMesh(core_axis_name="c", subcore_axis_name="s")

    @functools.partial(
        pl.kernel,
        mesh=mesh,
        out_type=jax.ShapeDtypeStruct((batch * row128, 128), jnp.float32),
        scratch_types=(
            [pltpu.VMEM((rows_per_w, seq), jnp.int32),
             pltpu.VMEM((seq * d,), jnp.float32)]
            + [pltpu.VMEM((seq, d), jnp.float32) for _ in range(NG)]
            + [pltpu.VMEM((row128, 128), jnp.float32) for _ in range(NO)]
            + [pltpu.SemaphoreType.DMA for _ in range(NG + NO)]
        ),
        compiler_params=pltpu.CompilerParams(use_tc_tiling_on_sc=False),
    )
    def emb(x_hbm, tok_hbm, pos_hbm, out_hbm, *scratch):
        idx_v = scratch[0]
        pos_v = scratch[1]
        gbufs = scratch[2:2 + NG]
        obufs = scratch[2 + NG:2 + NG + NO]
        gsems = scratch[2 + NG + NO:2 + NG + NO + NG]
        osems = scratch[2 + NG + NO + NG:]

        wid = lax.axis_index("s") * NC + lax.axis_index("c")
        base_row = wid * rows_per_w

        pltpu.sync_copy(x_hbm.at[pl.ds(base_row, rows_per_w)], idx_v)
        pltpu.sync_copy(pos_hbm, pos_v)

        def gather_start(r, slot):
            pltpu.make_async_copy(
                tok_hbm.at[idx_v.at[r]], gbufs[slot], gsems[slot]
            ).start()

        def gather_wait(r, slot):
            pltpu.make_async_copy(
                tok_hbm.at[idx_v.at[r]], gbufs[slot], gsems[slot]
            ).wait()

        def out_copy(r, slot):
            return pltpu.make_async_copy(
                obufs[slot],
                out_hbm.at[pl.ds((base_row + r) * row128, row128)],
                osems[slot],
            )

        grp = 128 // d

        for b in range(NG):
            gather_start(b, b)

        def group(g, _):
            r0 = g * NG
            for b in range(NG):
                r = r0 + b
                oslot = b % NO
                gather_wait(r, b)
                @pl.when(r >= NO)
                def _():
                    out_copy(r - NO, oslot).wait()

                def addrow(q, _):
                    for p in range(grp):
                        i = q * grp + p
                        for h in range(d // L):
                            tok = gbufs[b][i, pl.ds(h * L, L)]
                            pos = pos_v[pl.ds(i * d + h * L, L)]
                            obufs[oslot][q, pl.ds(p * d + h * L, L)] = tok + pos
                    return 0

                lax.fori_loop(0, row128, addrow, 0)
                out_copy(r, oslot).start()
                @pl.when(r + NG < rows_per_w)
                def _():
                    gather_start(r + NG, b)
            return 0

        lax.fori_loop(0, n_groups, group, 0)

        for b in range(NO):
            r = rows_per_w - NO + b
            out_copy(r, r % NO).wait()

    return emb


@functools.lru_cache(maxsize=None)
def _build_conv_out(batch, seq, d):
    assert d == D
    grp = 128 // d
    row128 = seq * d // 128
    BB = 128

    def body(i_ref, o_ref):
        a3 = i_ref[...].reshape(BB, row128, 128)
        for t in range(row128):
            o_ref[pl.ds(t * 128, 128), :] = a3[:, t, :].T

    return pl.pallas_call(
        body,
        grid=(batch // BB,),
        in_specs=[pl.BlockSpec((BB * row128, 128), lambda g: (g, 0))],
        out_specs=pl.BlockSpec((seq * d, BB), lambda g: (0, g)),
        out_shape=jax.ShapeDtypeStruct((seq * d, batch), jnp.float32),
    )


def kernel(x, token_table, position_table):
    batch, seq = x.shape
    vocab, d = token_table.shape
    pos_flat = position_table[:seq].reshape(-1)

    tok_lin = _build_conv_in(vocab, d)(token_table.T)
    tok_rows = tok_lin.reshape(vocab, d)
    out_sc = _build_emb(batch, seq, vocab, d)(x, tok_rows, pos_flat)
    out_t = _build_conv_out(batch, seq, d)(out_sc)
    return out_t.reshape(seq, d, batch).transpose(2, 0, 1)


# speedup vs baseline: 3.9267x; 3.9267x over previous
import functools

import jax
import jax.numpy as jnp
from jax import lax
from jax.experimental import pallas as pl
from jax.experimental.pallas import tpu as pltpu
from jax.experimental.pallas import tpu_sc as plsc

NC = 2
NS = 16
NW = NC * NS
L = 16

NG = 4
NO = 2
D = 32


@functools.lru_cache(maxsize=None)
def _build_conv_in(vocab, d):
    assert d == D
    grp = 128 // d
    VB = 512
    n_steps = -(-(vocab // grp) // VB)

    def body(t_ref, o_ref):
        a = t_ref[...]
        a3 = a.reshape(d, VB, grp)
        o_ref[...] = a3.transpose(1, 2, 0).reshape(VB, 128)

    return pl.pallas_call(
        body,
        grid=(n_steps,),
        in_specs=[pl.BlockSpec((d, grp * VB), lambda g: (0, g))],
        out_specs=pl.BlockSpec((VB, 128), lambda g: (g, 0)),
        out_shape=jax.ShapeDtypeStruct((vocab // grp, 128), jnp.float32),
    )


@functools.lru_cache(maxsize=None)
def _build_emb(batch, seq, vocab, d):
    assert d == D
    rows_per_w = batch // NW
    n_groups = rows_per_w // NG
    assert rows_per_w % NG == 0
    row128 = seq * d // 128
    mesh = plsc.VectorSubcoreMesh(core_axis_name="c", subcore_axis_name="s")

    @functools.partial(
        pl.kernel,
        mesh=mesh,
        out_type=jax.ShapeDtypeStruct((batch * row128, 128), jnp.float32),
        scratch_types=(
            [pltpu.VMEM((rows_per_w, seq), jnp.int32),
             pltpu.VMEM((seq * d,), jnp.float32)]
            + [pltpu.VMEM((seq, d), jnp.float32) for _ in range(NG)]
            + [pltpu.VMEM((row128, 128), jnp.float32) for _ in range(NO)]
            + [pltpu.SemaphoreType.DMA for _ in range(NG + NO)]
        ),
        compiler_params=pltpu.CompilerParams(use_tc_tiling_on_sc=False),
    )
    def emb(x_hbm, tok_hbm, pos_hbm, out_hbm, *scratch):
        idx_v = scratch[0]
        pos_v = scratch[1]
        gbufs = scratch[2:2 + NG]
        obufs = scratch[2 + NG:2 + NG + NO]
        gsems = scratch[2 + NG + NO:2 + NG + NO + NG]
        osems = scratch[2 + NG + NO + NG:]

        wid = lax.axis_index("s") * NC + lax.axis_index("c")
        base_row = wid * rows_per_w

        pltpu.sync_copy(x_hbm.at[pl.ds(base_row, rows_per_w)], idx_v)
        pltpu.sync_copy(pos_hbm, pos_v)

        def gather_start(r, slot):
            pltpu.make_async_copy(
                tok_hbm.at[idx_v.at[r]], gbufs[slot], gsems[slot]
            ).start()

        def gather_wait(r, slot):
            pltpu.make_async_copy(
                tok_hbm.at[idx_v.at[r]], gbufs[slot], gsems[slot]
            ).wait()

        def out_copy(r, slot):
            return pltpu.make_async_copy(
                obufs[slot],
                out_hbm.at[pl.ds((base_row + r) * row128, row128)],
                osems[slot],
            )

        grp = 128 // d

        for b in range(NG):
            gather_start(b, b)

        def group(g, _):
            r0 = g * NG
            for b in range(NG):
                r = r0 + b
                oslot = b % NO
                gather_wait(r, b)
                @pl.when(r >= NO)
                def _():
                    out_copy(r - NO, oslot).wait()

                def addrow(q, _):
                    for p in range(grp):
                        i = q * grp + p
                        for h in range(d // L):
                            tok = gbufs[b][i, pl.ds(h * L, L)]
                            pos = pos_v[pl.ds(i * d + h * L, L)]
                            obufs[oslot][q, pl.ds(p * d + h * L, L)] = tok + pos
                    return 0

                lax.fori_loop(0, row128, addrow, 0)
                out_copy(r, oslot).start()
                @pl.when(r + NG < rows_per_w)
                def _():
                    gather_start(r + NG, b)
            return 0

        lax.fori_loop(0, n_groups, group, 0)

        for b in range(NO):
            r = rows_per_w - NO + b
            out_copy(r, r % NO).wait()

    return emb


@functools.lru_cache(maxsize=None)
def _build_conv_out(batch, seq, d):
    assert d == D
    grp = 128 // d
    row128 = seq * d // 128
    BB = 128

    def body(i_ref, o_ref):
        a3 = i_ref[...].reshape(BB, row128, 128)
        for t in range(row128):
            o_ref[pl.ds(t * 128, 128), :] = a3[:, t, :].T

    return pl.pallas_call(
        body,
        grid=(batch // BB,),
        in_specs=[pl.BlockSpec((BB * row128, 128), lambda g: (g, 0))],
        out_specs=pl.BlockSpec((seq * d, BB), lambda g: (0, g)),
        out_shape=jax.ShapeDtypeStruct((seq * d, batch), jnp.float32),
    )


def kernel(x, token_table, position_table):
    batch, seq = x.shape
    vocab, d = token_table.shape
    pos_flat = position_table[:seq].reshape(-1)

    out_sc = _build_emb(batch, seq, vocab, d)(x, token_table, pos_flat)
    out_t = _build_conv_out(batch, seq, d)(out_sc)
    return out_t.reshape(seq, d, batch).transpose(2, 0, 1)


# speedup vs baseline: 4.4797x; 1.1408x over previous
import functools

import jax
import jax.numpy as jnp
from jax import lax
from jax.experimental import pallas as pl
from jax.experimental.pallas import tpu as pltpu
from jax.experimental.pallas import tpu_sc as plsc

NC = 2
NS = 16
NW = NC * NS
L = 16

NG = 4
NO = 2
D = 32


@functools.lru_cache(maxsize=None)
def _build_conv_in(vocab, d):
    assert d == D
    grp = 128 // d
    n_j = 16
    VB = n_j * 128 // grp
    n_steps = -(-(vocab // grp) // VB)

    def body(t_ref, o_ref):
        a3 = t_ref[...].reshape(d, n_j, 128)
        q_ids = lax.broadcasted_iota(jnp.int32, (128 // grp, 128), 0)
        u_ids = lax.broadcasted_iota(jnp.int32, (128 // grp, 128), 1)
        sel = [
            (u_ids == grp * q_ids + p).astype(jnp.float32)
            for p in range(grp)
        ]
        for j in range(n_j):
            a_j = a3[:, j, :]
            for p in range(grp):
                o_ref[pl.ds(j * 128 // grp, 128 // grp),
                      pl.ds(p * d, d)] = lax.dot_general(
                    sel[p], a_j, (((1,), (1,)), ((), ())),
                    preferred_element_type=jnp.float32)

    return pl.pallas_call(
        body,
        grid=(n_steps,),
        in_specs=[pl.BlockSpec((d, grp * VB), lambda g: (0, g))],
        out_specs=pl.BlockSpec((VB, 128), lambda g: (g, 0)),
        out_shape=jax.ShapeDtypeStruct((vocab // grp, 128), jnp.float32),
    )


@functools.lru_cache(maxsize=None)
def _build_emb(batch, seq, vocab, d):
    assert d == D
    rows_per_w = batch // NW
    n_groups = rows_per_w // NG
    assert rows_per_w % NG == 0
    row128 = seq * d // 128
    mesh = plsc.VectorSubcoreMesh(core_axis_name="c", subcore_axis_name="s")

    @functools.partial(
        pl.kernel,
        mesh=mesh,
        out_type=jax.ShapeDtypeStruct((batch * row128, 128), jnp.float32),
        scratch_types=(
            [pltpu.VMEM((rows_per_w, seq), jnp.int32),
             pltpu.VMEM((seq * d,), jnp.float32)]
            + [pltpu.VMEM((seq, d), jnp.float32) for _ in range(NG)]
            + [pltpu.VMEM((row128, 128), jnp.float32) for _ in range(NO)]
            + [pltpu.SemaphoreType.DMA for _ in range(NG + NO)]
        ),
        compiler_params=pltpu.CompilerParams(use_tc_tiling_on_sc=False),
    )
    def emb(x_hbm, tok_hbm, pos_hbm, out_hbm, *scratch):
        idx_v = scratch[0]
        pos_v = scratch[1]
        gbufs = scratch[2:2 + NG]
        obufs = scratch[2 + NG:2 + NG + NO]
        gsems = scratch[2 + NG + NO:2 + NG + NO + NG]
        osems = scratch[2 + NG + NO + NG:]

        wid = lax.axis_index("s") * NC + lax.axis_index("c")
        base_row = wid * rows_per_w

        pltpu.sync_copy(x_hbm.at[pl.ds(base_row, rows_per_w)], idx_v)
        pltpu.sync_copy(pos_hbm, pos_v)

        def gather_start(r, slot):
            pltpu.make_async_copy(
                tok_hbm.at[idx_v.at[r]], gbufs[slot], gsems[slot]
            ).start()

        def gather_wait(r, slot):
            pltpu.make_async_copy(
                tok_hbm.at[idx_v.at[r]], gbufs[slot], gsems[slot]
            ).wait()

        def out_copy(r, slot):
            return pltpu.make_async_copy(
                obufs[slot],
                out_hbm.at[pl.ds((base_row + r) * row128, row128)],
                osems[slot],
            )

        grp = 128 // d

        for b in range(NG):
            gather_start(b, b)

        def group(g, _):
            r0 = g * NG
            for b in range(NG):
                r = r0 + b
                oslot = b % NO
                gather_wait(r, b)
                @pl.when(r >= NO)
                def _():
                    out_copy(r - NO, oslot).wait()

                def addrow(q, _):
                    for p in range(grp):
                        i = q * grp + p
                        for h in range(d // L):
                            tok = gbufs[b][i, pl.ds(h * L, L)]
                            pos = pos_v[pl.ds(i * d + h * L, L)]
                            obufs[oslot][q, pl.ds(p * d + h * L, L)] = tok + pos
                    return 0

                lax.fori_loop(0, row128, addrow, 0)
                out_copy(r, oslot).start()
                @pl.when(r + NG < rows_per_w)
                def _():
                    gather_start(r + NG, b)
            return 0

        lax.fori_loop(0, n_groups, group, 0)

        for b in range(NO):
            r = rows_per_w - NO + b
            out_copy(r, r % NO).wait()

    return emb


@functools.lru_cache(maxsize=None)
def _build_conv_out(batch, seq, d):
    assert d == D
    grp = 128 // d
    row128 = seq * d // 128
    BB = 128

    def body(i_ref, o_ref):
        a3 = i_ref[...].reshape(BB, row128, 128)
        for t in range(row128):
            o_ref[pl.ds(t * 128, 128), :] = a3[:, t, :].T

    return pl.pallas_call(
        body,
        grid=(batch // BB,),
        in_specs=[pl.BlockSpec((BB * row128, 128), lambda g: (g, 0))],
        out_specs=pl.BlockSpec((seq * d, BB), lambda g: (0, g)),
        out_shape=jax.ShapeDtypeStruct((seq * d, batch), jnp.float32),
    )


def kernel(x, token_table, position_table):
    batch, seq = x.shape
    vocab, d = token_table.shape
    pos_flat = position_table[:seq].reshape(-1)

    tok_lin = _build_conv_in(vocab, d)(token_table.T)
    tok_rows = tok_lin.reshape(vocab, d)
    out_sc = _build_emb(batch, seq, vocab, d)(x, tok_rows, pos_flat)
    out_t = _build_conv_out(batch, seq, d)(out_sc)
    return out_t.reshape(seq, d, batch).transpose(2, 0, 1)
